# baseline (speedup 1.0000x reference)
import functools

import numpy as np
import jax
import jax.numpy as jnp
from jax import lax
from jax.experimental import pallas as pl
from jax.experimental.pallas import tpu as pltpu
from jax.experimental.pallas import tpu_sc as plsc

_SCALES = (0.1, 0.25, 0.5)
_K = 16
_DE = 64
_BM = 256
_BN = 256

_INV_SQRT2 = float(1.0 / np.sqrt(2.0))


def _gelu(z):
    return 0.5 * z * (1.0 + lax.erf(z * _INV_SQRT2))


def _select_body(gpos_ref, ppos_ref,
                 idxphi_ref, cntphi_ref, idxpsi_ref, cntpsi_ref,
                 cnt1_ref, acc_ref, cnt2_ref, *, nbn, nbm, r2s):
    nb = pl.program_id(1)
    mb = pl.program_id(2)
    kf = float(_K)

    @pl.when(nb == 0)
    def _():
        cnt1_ref[:, pl.ds(mb * _BM, _BM), :] = jnp.zeros((3, _BM, 1), jnp.float32)
        acc_ref[:, pl.ds(mb * _BM, _BM), :] = jnp.zeros((3, _BM, _K), jnp.float32)

    @pl.when(mb == 0)
    def _():
        cnt2_ref[...] = jnp.zeros_like(cnt2_ref)
        idxpsi_ref[...] = jnp.zeros_like(idxpsi_ref)

    done1 = jnp.min(cnt1_ref[:, pl.ds(mb * _BM, _BM), :]) >= kf
    done2 = jnp.min(cnt2_ref[...]) >= kf

    @pl.when(jnp.logical_not(jnp.logical_and(done1, done2)))
    def _():
        gp = gpos_ref[0]
        cx, cy = gp[:, 0:1], gp[:, 1:2]
        pp = ppos_ref[0]
        px, py = pp[0:1, :], pp[1:2, :]
        c2 = cx * cx + cy * cy
        p2 = px * px + py * py
        cxb = cx.astype(jnp.bfloat16).astype(jnp.float32)
        cyb = cy.astype(jnp.bfloat16).astype(jnp.float32)
        pxb = px.astype(jnp.bfloat16).astype(jnp.float32)
        pyb = py.astype(jnp.bfloat16).astype(jnp.float32)
        d2 = jnp.maximum((c2 + p2) - 2.0 * (cxb * pxb + cyb * pyb), 0.0)

        rowi = lax.broadcasted_iota(jnp.int32, (_BM, _BN), 0).astype(jnp.float32)
        coli = lax.broadcasted_iota(jnp.int32, (_BM, _BN), 1).astype(jnp.float32)
        n_glob = coli + float(_BN) * nb.astype(jnp.float32)
        m_glob = rowi + float(_BM) * mb.astype(jnp.float32)
        tu = (rowi <= coli).astype(jnp.float32)
        tl = (coli <= rowi).astype(jnp.float32)

        for s, r in enumerate(r2s):
            within = d2 <= r
            wf = within.astype(jnp.float32)

            prev1 = cnt1_ref[s, pl.ds(mb * _BM, _BM), :]

            @pl.when(jnp.min(prev1) < kf)
            def _(within=within, wf=wf, prev1=prev1, s=s):
                rank = prev1 + jnp.dot(wf, tu, preferred_element_type=jnp.float32)
                sel = within & (rank <= kf)
                val = jnp.where(sel, n_glob, 0.0)
                rnk = jnp.where(sel, rank, 0.0)
                new1 = jnp.sum(wf, axis=1, keepdims=True)
                jlo = jnp.min(prev1)
                jhi = jnp.max(prev1 + new1)
                for j in range(_K):
                    @pl.when(jnp.logical_and(float(j + 1) > jlo,
                                             float(j + 1) <= jhi))
                    def _(j=j):
                        col = jnp.sum(jnp.where(rnk == float(j + 1), val, 0.0),
                                      axis=1, keepdims=True)
                        acc_ref[s, pl.ds(mb * _BM, _BM), pl.ds(j, 1)] += col
                cnt1_ref[s, pl.ds(mb * _BM, _BM), :] = prev1 + new1

            prev2 = cnt2_ref[s]

            @pl.when(jnp.min(prev2) < kf)
            def _(within=within, wf=wf, prev2=prev2, s=s):
                rank = prev2 + jnp.dot(tl, wf, preferred_element_type=jnp.float32)
                sel = within & (rank <= kf)
                val = jnp.where(sel, m_glob, 0.0)
                rnk = jnp.where(sel, rank, 0.0)
                new2 = jnp.sum(wf, axis=0, keepdims=True)
                jlo = jnp.min(prev2)
                jhi = jnp.max(prev2 + new2)
                for j in range(_K):
                    @pl.when(jnp.logical_and(float(j + 1) > jlo,
                                             float(j + 1) <= jhi))
                    def _(j=j):
                        row = jnp.sum(jnp.where(rnk == float(j + 1), val, 0.0),
                                      axis=0, keepdims=True)
                        idxpsi_ref[0, s, pl.ds(j, 1), :] += row
                cnt2_ref[s] = prev2 + new2

    @pl.when(nb == nbn - 1)
    def _():
        idxphi_ref[0] = acc_ref[:, pl.ds(mb * _BM, _BM), :]
        cntphi_ref[0] = jnp.minimum(cnt1_ref[:, pl.ds(mb * _BM, _BM), :], kf)

    @pl.when(mb == nbm - 1)
    def _():
        cntpsi_ref[0] = jnp.minimum(cnt2_ref[...], kf)


def _select(gpos, ppos_t):
    B, M, _ = gpos.shape
    N = ppos_t.shape[2]
    nbn, nbm = N // _BN, M // _BM
    r2s = tuple(float(np.float32(r * r)) for r in _SCALES)
    out_shapes = (
        jax.ShapeDtypeStruct((B, 3, M, _K), jnp.float32),
        jax.ShapeDtypeStruct((B, 3, M, 1), jnp.float32),
        jax.ShapeDtypeStruct((B, 3, _K, N), jnp.float32),
        jax.ShapeDtypeStruct((B, 3, 1, N), jnp.float32),
    )
    return pl.pallas_call(
        functools.partial(_select_body, nbn=nbn, nbm=nbm, r2s=r2s),
        grid=(B, nbn, nbm),
        in_specs=[
            pl.BlockSpec((1, _BM, 2), lambda b, nb, mb: (b, mb, 0)),
            pl.BlockSpec((1, 2, _BN), lambda b, nb, mb: (b, 0, nb)),
        ],
        out_specs=[
            pl.BlockSpec((1, 3, _BM, _K), lambda b, nb, mb: (b, 0, mb, 0)),
            pl.BlockSpec((1, 3, _BM, 1), lambda b, nb, mb: (b, 0, mb, 0)),
            pl.BlockSpec((1, 3, _K, _BN), lambda b, nb, mb: (b, 0, 0, nb)),
            pl.BlockSpec((1, 3, 1, _BN), lambda b, nb, mb: (b, 0, 0, nb)),
        ],
        out_shape=out_shapes,
        scratch_shapes=[
            pltpu.VMEM((3, M, 1), jnp.float32),
            pltpu.VMEM((3, M, _K), jnp.float32),
            pltpu.VMEM((3, 1, _BN), jnp.float32),
        ],
    )(gpos, ppos_t)


def _sc_gather(table, idx, chunk=2048):
    R = idx.shape[0]
    width = table.shape[1]
    nw = 32
    per_w = R // nw
    nch = per_w // chunk
    mesh = plsc.VectorSubcoreMesh(core_axis_name="c", subcore_axis_name="s")

    @functools.partial(
        pl.kernel,
        out_type=jax.ShapeDtypeStruct((R, width), jnp.float32),
        mesh=mesh,
        compiler_params=pltpu.CompilerParams(use_tc_tiling_on_sc=False),
        scratch_types=[
            pltpu.VMEM((chunk,), jnp.int32),
            pltpu.VMEM((chunk, width), jnp.float32),
            pltpu.SemaphoreType.DMA,
        ],
    )
    def gk(tab_hbm, idx_hbm, out_hbm, idx_v, rows_v, sem):
        wid = lax.axis_index("s") * 2 + lax.axis_index("c")
        base = pl.multiple_of(wid * per_w, 8)
        for c in range(nch):
            off = pl.multiple_of(base + c * chunk, 8)
            pltpu.sync_copy(idx_hbm.at[pl.ds(off, chunk)], idx_v)
            pltpu.async_copy(tab_hbm.at[idx_v], rows_v, sem).wait()
            pltpu.sync_copy(rows_v, out_hbm.at[pl.ds(off, chunk)])

    return gk(table, idx)


def _pool_body(gath_ref, ctr_ref, cnt_ref, w1_ref, b1_ref, w2_ref, b2_ref,
               out_ref, *, din, cm, accumulate):
    g = gath_ref[0, 0]
    g3 = g.reshape(cm, _K, 16)
    ctr = ctr_ref[0]
    rel = g3[:, :, din:din + 2] - ctr[:, None, :]
    x = jnp.concatenate([g3[:, :, 0:din], rel], axis=-1).reshape(cm * _K, din + 2)
    z = jnp.dot(x, w1_ref[0], preferred_element_type=jnp.float32) + b1_ref[0]
    h = jnp.dot(_gelu(z), w2_ref[0], preferred_element_type=jnp.float32) + b2_ref[0]
    cnt = cnt_ref[0, 0]
    kio = lax.broadcasted_iota(jnp.int32, (cm, _K), 1).astype(jnp.float32)
    msk = (kio < cnt).astype(jnp.float32)
    h3 = h.reshape(cm, _K, _DE)
    hm = jnp.sum(h3 * msk[:, :, None], axis=1) / jnp.maximum(cnt, 1.0)

    if accumulate:
        @pl.when(pl.program_id(2) == 0)
        def _():
            out_ref[0, 0] = jnp.zeros_like(out_ref[0, 0])
        out_ref[0, 0] += jnp.sum(hm, axis=0, keepdims=True)
    else:
        out_ref[0, 0] = hm


def _pool(gath, centers, cnt, w1, b1, w2, b2, din, accumulate, cm=256):
    B = gath.shape[0]
    R = gath.shape[2] // _K
    nc = R // cm
    if accumulate:
        out_shape = jax.ShapeDtypeStruct((B, 3, 1, _DE), jnp.float32)
        out_spec = pl.BlockSpec((1, 1, 1, _DE), lambda b, s, c: (b, s, 0, 0))
    else:
        out_shape = jax.ShapeDtypeStruct((B, 3, R, _DE), jnp.float32)
        out_spec = pl.BlockSpec((1, 1, cm, _DE), lambda b, s, c: (b, s, c, 0))
    return pl.pallas_call(
        functools.partial(_pool_body, din=din, cm=cm, accumulate=accumulate),
        grid=(B, 3, nc),
        in_specs=[
            pl.BlockSpec((1, 1, cm * _K, 16), lambda b, s, c: (b, s, c, 0)),
            pl.BlockSpec((1, cm, 2), lambda b, s, c: (b, c, 0)),
            pl.BlockSpec((1, 1, cm, 1), lambda b, s, c: (b, s, c, 0)),
            pl.BlockSpec((1, din + 2, _DE), lambda b, s, c: (s, 0, 0)),
            pl.BlockSpec((1, 1, _DE), lambda b, s, c: (s, 0, 0)),
            pl.BlockSpec((1, _DE, _DE), lambda b, s, c: (s, 0, 0)),
            pl.BlockSpec((1, 1, _DE), lambda b, s, c: (s, 0, 0)),
        ],
        out_specs=out_spec,
        out_shape=out_shape,
    )(gath, centers, cnt, w1, b1, w2, b2)


def _head_body(gp_ref, gw1_ref, gb1_ref, gw2_ref, gb2_ref,
               gf_ref, ew1_ref, eb1_ref, ew2_ref, eb2_ref,
               es_ref, pw_ref, pb_ref, out_ref):
    B, Mg, _ = gf_ref.shape

    def mlp(x, w1, b1, w2, b2):
        z = jnp.dot(x, w1, preferred_element_type=jnp.float32) + b1
        return jnp.dot(_gelu(z), w2, preferred_element_type=jnp.float32) + b2

    p_enc = mlp(gp_ref[...], gw1_ref[...], gb1_ref[...], gw2_ref[...], gb2_ref[...])
    ge = mlp(gf_ref[...].reshape(B * Mg, 2), ew1_ref[...], eb1_ref[...],
             ew2_ref[...], eb2_ref[...])
    c_geom = jnp.mean(ge.reshape(B, Mg, _DE), axis=1)
    raw = jnp.concatenate([p_enc, c_geom, es_ref[...]], axis=1)
    out_ref[...] = jnp.dot(raw, pw_ref[...], preferred_element_type=jnp.float32) + pb_ref[...]


def _head(global_params, gw1, gb1, gw2, gb2, geometry_features,
          ew1, eb1, ew2, eb2, e_flat, proj_w, proj_b):
    B = global_params.shape[0]
    return pl.pallas_call(
        _head_body,
        out_shape=jax.ShapeDtypeStruct((B, proj_w.shape[1]), jnp.float32),
    )(global_params, gw1, gb1.reshape(1, -1), gw2, gb2.reshape(1, -1),
      geometry_features, ew1, eb1.reshape(1, -1), ew2, eb2.reshape(1, -1),
      e_flat, proj_w, proj_b.reshape(1, -1))


def kernel(global_params, geometry_positions, geometry_features,
           input_positions, input_features,
           gw1, gb1, gw2, gb2, ew1, eb1, ew2, eb2,
           phi_w1, phi_b1, phi_w2, phi_b2,
           psi_w1, psi_b1, psi_w2, psi_b2,
           proj_w, proj_b):
    f32 = jnp.float32
    B, M, _ = geometry_positions.shape
    N = input_positions.shape[1]

    ppos_t = jnp.transpose(input_positions, (0, 2, 1))
    idx_phi_f, cnt_phi, idx_psi_f, cnt_psi_t = _select(geometry_positions, ppos_t)

    idx_phi = idx_phi_f.astype(jnp.int32)
    idx_psi = jnp.transpose(idx_psi_f, (0, 1, 3, 2)).astype(jnp.int32)
    cnt_psi = jnp.transpose(cnt_psi_t, (0, 1, 3, 2))

    boff = (jnp.arange(B, dtype=jnp.int32) * N).reshape(B, 1, 1, 1)
    tab_in = jnp.concatenate(
        [input_features, input_positions, jnp.zeros((B, N, 6), f32)],
        axis=-1).reshape(B * N, 16)
    gath_phi = _sc_gather(tab_in, (idx_phi + boff).reshape(-1)
                          ).reshape(B, 3, M * _K, 16)

    goff = (jnp.arange(B, dtype=jnp.int32) * M).reshape(B, 1, 1, 1)
    tab_g = jnp.concatenate(
        [geometry_features, geometry_positions, jnp.zeros((B, M, 12), f32)],
        axis=-1).reshape(B * M, 16)
    gath_psi = _sc_gather(tab_g, (idx_psi + goff).reshape(-1)
                          ).reshape(B, 3, N * _K, 16)

    e_sums = _pool(gath_phi, geometry_positions, cnt_phi,
                   phi_w1, phi_b1.reshape(3, 1, _DE),
                   phi_w2, phi_b2.reshape(3, 1, _DE), 8, True)
    aug_s = _pool(gath_psi, input_positions, cnt_psi,
                  psi_w1, psi_b1.reshape(3, 1, _DE),
                  psi_w2, psi_b2.reshape(3, 1, _DE), 2, False)

    augmentation = jnp.concatenate([aug_s[:, 0], aug_s[:, 1], aug_s[:, 2]],
                                   axis=-1)
    e_flat = (e_sums[:, :, 0, :] / float(M)).reshape(B, 3 * _DE)
    context = _head(global_params, gw1, gb1, gw2, gb2, geometry_features,
                    ew1, eb1, ew2, eb2, e_flat, proj_w, proj_b)
    return context, augmentation


# speedup vs baseline: 1.0771x; 1.0771x over previous
import functools

import numpy as np
import jax
import jax.numpy as jnp
from jax import lax
from jax.experimental import pallas as pl
from jax.experimental.pallas import tpu as pltpu
from jax.experimental.pallas import tpu_sc as plsc

_SCALES = (0.1, 0.25, 0.5)
_K = 16
_DE = 64
_BM = 256
_BN = 256

_INV_SQRT2 = float(1.0 / np.sqrt(2.0))


def _gelu(z):
    return 0.5 * z * (1.0 + lax.erf(z * _INV_SQRT2))


def _select_body(gpos_ref, ppos_ref,
                 idxphi_ref, cntphi_ref, idxpsi_ref, cntpsi_ref,
                 cnt1_ref, cnt2_ref, *, nbn, nbm, r2s):
    nb = pl.program_id(1)
    mb = pl.program_id(2)
    kf = float(_K)

    @pl.when(nb == 0)
    def _():
        cnt1_ref[:, pl.ds(mb * _BM, _BM), :] = jnp.zeros((3, _BM, 1), jnp.float32)
        idxphi_ref[0, :, pl.ds(mb * _BM, _BM), :] = jnp.zeros(
            (3, _BM, _K), jnp.float32)

    @pl.when(mb == 0)
    def _():
        cnt2_ref[...] = jnp.zeros_like(cnt2_ref)
        idxpsi_ref[0, :, :, pl.ds(nb * _BN, _BN)] = jnp.zeros(
            (3, _K, _BN), jnp.float32)

    done1 = jnp.min(cnt1_ref[:, pl.ds(mb * _BM, _BM), :]) >= kf
    done2 = jnp.min(cnt2_ref[...]) >= kf

    @pl.when(jnp.logical_not(jnp.logical_and(done1, done2)))
    def _():
        gp = gpos_ref[0]
        cx, cy = gp[:, 0:1], gp[:, 1:2]
        pp = ppos_ref[0]
        px, py = pp[0:1, :], pp[1:2, :]
        c2 = cx * cx + cy * cy
        p2 = px * px + py * py
        cxb = cx.astype(jnp.bfloat16).astype(jnp.float32)
        cyb = cy.astype(jnp.bfloat16).astype(jnp.float32)
        pxb = px.astype(jnp.bfloat16).astype(jnp.float32)
        pyb = py.astype(jnp.bfloat16).astype(jnp.float32)
        d2 = jnp.maximum((c2 + p2) - 2.0 * (cxb * pxb + cyb * pyb), 0.0)

        rowi = lax.broadcasted_iota(jnp.int32, (_BM, _BN), 0).astype(jnp.float32)
        coli = lax.broadcasted_iota(jnp.int32, (_BM, _BN), 1).astype(jnp.float32)
        n_glob = coli + float(_BN) * nb.astype(jnp.float32)
        m_glob = rowi + float(_BM) * mb.astype(jnp.float32)
        tu = (rowi <= coli).astype(jnp.float32)
        tl = (coli <= rowi).astype(jnp.float32)

        for s, r in enumerate(r2s):
            within = d2 <= r
            wf = within.astype(jnp.float32)

            prev1 = cnt1_ref[s, pl.ds(mb * _BM, _BM), :]

            @pl.when(jnp.min(prev1) < kf)
            def _(within=within, wf=wf, prev1=prev1, s=s):
                rank = prev1 + jnp.dot(wf, tu, preferred_element_type=jnp.float32)
                sel = within & (rank <= kf)
                val = jnp.where(sel, n_glob, 0.0)
                rnk = jnp.where(sel, rank, 0.0)
                cols = [jnp.sum(jnp.where(rnk == float(j + 1), val, 0.0),
                                axis=1, keepdims=True) for j in range(_K)]
                idxphi_ref[0, s, pl.ds(mb * _BM, _BM), :] += jnp.concatenate(
                    cols, axis=1)
                cnt1_ref[s, pl.ds(mb * _BM, _BM), :] = (
                    prev1 + jnp.sum(wf, axis=1, keepdims=True))

            prev2 = cnt2_ref[s]

            @pl.when(jnp.min(prev2) < kf)
            def _(within=within, wf=wf, prev2=prev2, s=s):
                rank = prev2 + jnp.dot(tl, wf, preferred_element_type=jnp.float32)
                sel = within & (rank <= kf)
                val = jnp.where(sel, m_glob, 0.0)
                rnk = jnp.where(sel, rank, 0.0)
                rows = [jnp.sum(jnp.where(rnk == float(j + 1), val, 0.0),
                                axis=0, keepdims=True) for j in range(_K)]
                idxpsi_ref[0, s, :, pl.ds(nb * _BN, _BN)] += jnp.concatenate(
                    rows, axis=0)
                cnt2_ref[s] = prev2 + jnp.sum(wf, axis=0, keepdims=True)

    @pl.when(nb == nbn - 1)
    def _():
        cntphi_ref[0, :, pl.ds(mb * _BM, _BM), :] = jnp.minimum(
            cnt1_ref[:, pl.ds(mb * _BM, _BM), :], kf)

    @pl.when(mb == nbm - 1)
    def _():
        cntpsi_ref[0, :, :, pl.ds(nb * _BN, _BN)] = jnp.minimum(
            cnt2_ref[...], kf)


def _select(gpos, ppos_t):
    B, M, _ = gpos.shape
    N = ppos_t.shape[2]
    nbn, nbm = N // _BN, M // _BM
    r2s = tuple(float(np.float32(r * r)) for r in _SCALES)
    out_shapes = (
        jax.ShapeDtypeStruct((B, 3, M, _K), jnp.float32),
        jax.ShapeDtypeStruct((B, 3, M, 1), jnp.float32),
        jax.ShapeDtypeStruct((B, 3, _K, N), jnp.float32),
        jax.ShapeDtypeStruct((B, 3, 1, N), jnp.float32),
    )
    return pl.pallas_call(
        functools.partial(_select_body, nbn=nbn, nbm=nbm, r2s=r2s),
        grid=(B, nbn, nbm),
        in_specs=[
            pl.BlockSpec((1, _BM, 2), lambda b, nb, mb: (b, mb, 0)),
            pl.BlockSpec((1, 2, _BN), lambda b, nb, mb: (b, 0, nb)),
        ],
        out_specs=[
            pl.BlockSpec((1, 3, M, _K), lambda b, nb, mb: (b, 0, 0, 0)),
            pl.BlockSpec((1, 3, M, 1), lambda b, nb, mb: (b, 0, 0, 0)),
            pl.BlockSpec((1, 3, _K, N), lambda b, nb, mb: (b, 0, 0, 0)),
            pl.BlockSpec((1, 3, 1, N), lambda b, nb, mb: (b, 0, 0, 0)),
        ],
        out_shape=out_shapes,
        scratch_shapes=[
            pltpu.VMEM((3, M, 1), jnp.float32),
            pltpu.VMEM((3, 1, _BN), jnp.float32),
        ],
    )(gpos, ppos_t)


def _sc_gather(table, idx, chunk=2048):
    R = idx.shape[0]
    width = table.shape[1]
    nw = 32
    per_w = R // nw
    nch = per_w // chunk
    mesh = plsc.VectorSubcoreMesh(core_axis_name="c", subcore_axis_name="s")

    @functools.partial(
        pl.kernel,
        out_type=jax.ShapeDtypeStruct((R, width), jnp.float32),
        mesh=mesh,
        compiler_params=pltpu.CompilerParams(use_tc_tiling_on_sc=False),
        scratch_types=[
            pltpu.VMEM((chunk,), jnp.int32),
            pltpu.VMEM((chunk, width), jnp.float32),
            pltpu.SemaphoreType.DMA,
        ],
    )
    def gk(tab_hbm, idx_hbm, out_hbm, idx_v, rows_v, sem):
        wid = lax.axis_index("s") * 2 + lax.axis_index("c")
        base = pl.multiple_of(wid * per_w, 8)
        for c in range(nch):
            off = pl.multiple_of(base + c * chunk, 8)
            pltpu.sync_copy(idx_hbm.at[pl.ds(off, chunk)], idx_v)
            pltpu.async_copy(tab_hbm.at[idx_v], rows_v, sem).wait()
            pltpu.sync_copy(rows_v, out_hbm.at[pl.ds(off, chunk)])

    return gk(table, idx)


def _pool_body(gath_ref, ctr_ref, cnt_ref, w1_ref, b1_ref, w2_ref, b2_ref,
               out_ref, *, din, cm, accumulate):
    g = gath_ref[0, 0]
    g3 = g.reshape(cm, _K, 16)
    ctr = ctr_ref[0]
    rel = g3[:, :, din:din + 2] - ctr[:, None, :]
    x = jnp.concatenate([g3[:, :, 0:din], rel], axis=-1).reshape(cm * _K, din + 2)
    z = jnp.dot(x, w1_ref[0], preferred_element_type=jnp.float32) + b1_ref[0]
    h = jnp.dot(_gelu(z), w2_ref[0], preferred_element_type=jnp.float32) + b2_ref[0]
    cnt = cnt_ref[0, 0]
    kio = lax.broadcasted_iota(jnp.int32, (cm, _K), 1).astype(jnp.float32)
    msk = (kio < cnt).astype(jnp.float32)
    h3 = h.reshape(cm, _K, _DE)
    hm = jnp.sum(h3 * msk[:, :, None], axis=1) / jnp.maximum(cnt, 1.0)

    if accumulate:
        @pl.when(pl.program_id(2) == 0)
        def _():
            out_ref[0, 0] = jnp.zeros_like(out_ref[0, 0])
        out_ref[0, 0] += jnp.sum(hm, axis=0, keepdims=True)
    else:
        out_ref[0, 0] = hm


def _pool(gath, centers, cnt, w1, b1, w2, b2, din, accumulate, cm=256):
    B = gath.shape[0]
    R = gath.shape[2] // _K
    nc = R // cm
    if accumulate:
        out_shape = jax.ShapeDtypeStruct((B, 3, 1, _DE), jnp.float32)
        out_spec = pl.BlockSpec((1, 1, 1, _DE), lambda b, s, c: (b, s, 0, 0))
    else:
        out_shape = jax.ShapeDtypeStruct((B, 3, R, _DE), jnp.float32)
        out_spec = pl.BlockSpec((1, 1, cm, _DE), lambda b, s, c: (b, s, c, 0))
    return pl.pallas_call(
        functools.partial(_pool_body, din=din, cm=cm, accumulate=accumulate),
        grid=(B, 3, nc),
        in_specs=[
            pl.BlockSpec((1, 1, cm * _K, 16), lambda b, s, c: (b, s, c, 0)),
            pl.BlockSpec((1, cm, 2), lambda b, s, c: (b, c, 0)),
            pl.BlockSpec((1, 1, cm, 1), lambda b, s, c: (b, s, c, 0)),
            pl.BlockSpec((1, din + 2, _DE), lambda b, s, c: (s, 0, 0)),
            pl.BlockSpec((1, 1, _DE), lambda b, s, c: (s, 0, 0)),
            pl.BlockSpec((1, _DE, _DE), lambda b, s, c: (s, 0, 0)),
            pl.BlockSpec((1, 1, _DE), lambda b, s, c: (s, 0, 0)),
        ],
        out_specs=out_spec,
        out_shape=out_shape,
    )(gath, centers, cnt, w1, b1, w2, b2)


def _head_body(gp_ref, gw1_ref, gb1_ref, gw2_ref, gb2_ref,
               gf_ref, ew1_ref, eb1_ref, ew2_ref, eb2_ref,
               es_ref, pw_ref, pb_ref, out_ref):
    B, Mg, _ = gf_ref.shape

    def mlp(x, w1, b1, w2, b2):
        z = jnp.dot(x, w1, preferred_element_type=jnp.float32) + b1
        return jnp.dot(_gelu(z), w2, preferred_element_type=jnp.float32) + b2

    p_enc = mlp(gp_ref[...], gw1_ref[...], gb1_ref[...], gw2_ref[...], gb2_ref[...])
    ge = mlp(gf_ref[...].reshape(B * Mg, 2), ew1_ref[...], eb1_ref[...],
             ew2_ref[...], eb2_ref[...])
    c_geom = jnp.mean(ge.reshape(B, Mg, _DE), axis=1)
    raw = jnp.concatenate([p_enc, c_geom, es_ref[...]], axis=1)
    out_ref[...] = jnp.dot(raw, pw_ref[...], preferred_element_type=jnp.float32) + pb_ref[...]


def _head(global_params, gw1, gb1, gw2, gb2, geometry_features,
          ew1, eb1, ew2, eb2, e_flat, proj_w, proj_b):
    B = global_params.shape[0]
    return pl.pallas_call(
        _head_body,
        out_shape=jax.ShapeDtypeStruct((B, proj_w.shape[1]), jnp.float32),
    )(global_params, gw1, gb1.reshape(1, -1), gw2, gb2.reshape(1, -1),
      geometry_features, ew1, eb1.reshape(1, -1), ew2, eb2.reshape(1, -1),
      e_flat, proj_w, proj_b.reshape(1, -1))


def kernel(global_params, geometry_positions, geometry_features,
           input_positions, input_features,
           gw1, gb1, gw2, gb2, ew1, eb1, ew2, eb2,
           phi_w1, phi_b1, phi_w2, phi_b2,
           psi_w1, psi_b1, psi_w2, psi_b2,
           proj_w, proj_b):
    f32 = jnp.float32
    B, M, _ = geometry_positions.shape
    N = input_positions.shape[1]

    ppos_t = jnp.transpose(input_positions, (0, 2, 1))
    idx_phi_f, cnt_phi, idx_psi_f, cnt_psi_t = _select(geometry_positions, ppos_t)

    idx_phi = idx_phi_f.astype(jnp.int32)
    idx_psi = jnp.transpose(idx_psi_f, (0, 1, 3, 2)).astype(jnp.int32)
    cnt_psi = jnp.transpose(cnt_psi_t, (0, 1, 3, 2))

    boff = (jnp.arange(B, dtype=jnp.int32) * N).reshape(B, 1, 1, 1)
    tab_in = jnp.concatenate(
        [input_features, input_positions, jnp.zeros((B, N, 6), f32)],
        axis=-1).reshape(B * N, 16)
    gath_phi = _sc_gather(tab_in, (idx_phi + boff).reshape(-1)
                          ).reshape(B, 3, M * _K, 16)

    goff = (jnp.arange(B, dtype=jnp.int32) * M).reshape(B, 1, 1, 1)
    tab_g = jnp.concatenate(
        [geometry_features, geometry_positions, jnp.zeros((B, M, 12), f32)],
        axis=-1).reshape(B * M, 16)
    gath_psi = _sc_gather(tab_g, (idx_psi + goff).reshape(-1)
                          ).reshape(B, 3, N * _K, 16)

    e_sums = _pool(gath_phi, geometry_positions, cnt_phi,
                   phi_w1, phi_b1.reshape(3, 1, _DE),
                   phi_w2, phi_b2.reshape(3, 1, _DE), 8, True)
    aug_s = _pool(gath_psi, input_positions, cnt_psi,
                  psi_w1, psi_b1.reshape(3, 1, _DE),
                  psi_w2, psi_b2.reshape(3, 1, _DE), 2, False)

    augmentation = jnp.concatenate([aug_s[:, 0], aug_s[:, 1], aug_s[:, 2]],
                                   axis=-1)
    e_flat = (e_sums[:, :, 0, :] / float(M)).reshape(B, 3 * _DE)
    context = _head(global_params, gw1, gb1, gw2, gb2, geometry_features,
                    ew1, eb1, ew2, eb2, e_flat, proj_w, proj_b)
    return context, augmentation
